# baseline (speedup 1.0000x reference)
import functools

import jax
import jax.numpy as jnp
from jax import lax
from jax.experimental import pallas as pl
from jax.experimental.pallas import tpu as pltpu
from jax.experimental.pallas import tpu_sc as plsc

N_NODES = 10000
N_EDGES = 160000
NC, NS = 2, 16
NW = NC * NS
K = 64
CH = 80
EPT = CH * K
EPW = N_EDGES // NW
NBUF = 1
NA = 10112
RPT = NA // NS

BM = 1000


def _worker(c, s):
    return s * NC + c


def _deg_body(dst_hbm, ones_hbm, zeros_hbm, out_hbm, idx_v, ones_v, accum, sem):
    c = lax.axis_index("c")
    s = lax.axis_index("s")
    wid = _worker(c, s)
    pltpu.sync_copy(dst_hbm.at[wid], idx_v)
    pltpu.sync_copy(ones_hbm, ones_v)
    pltpu.sync_copy(zeros_hbm.at[pl.ds(s * RPT, RPT)],
                    accum.at[pl.ds(s * RPT, RPT)])
    plsc.subcore_barrier()

    @pl.loop(0, CH, step=8)
    def _grp(i):
        for b in range(8):
            pltpu.async_copy(ones_v, accum.at[idx_v.at[i + b]], sem, add=True)
        for b in range(8):
            pltpu.make_async_copy(ones_v, accum.at[idx_v.at[i + b]], sem).wait()

    plsc.subcore_barrier()
    pltpu.sync_copy(accum.at[pl.ds(s * RPT, RPT)],
                    out_hbm.at[c, pl.ds(s * RPT, RPT)])


_deg_kernel = pl.kernel(
    _deg_body,
    out_type=jax.ShapeDtypeStruct((NC, NA, 16), jnp.float32),
    mesh=plsc.VectorSubcoreMesh(core_axis_name="c", subcore_axis_name="s",
                                num_cores=NC, num_subcores=NS),
    compiler_params=pltpu.CompilerParams(use_tc_tiling_on_sc=False),
    scratch_types=[
        pltpu.VMEM((CH, K), jnp.int32),
        pltpu.VMEM((K, 16), jnp.float32),
        pltpu.VMEM_SHARED((NA, 16), jnp.float32),
        pltpu.SemaphoreType.DMA,
    ],
)


def _prop_body(fw, npair, *refs):
    ys = refs[:npair]
    src_hbm, dst_hbm, zeros_hbm = refs[npair:npair + 3]
    outs = refs[npair + 3:2 * npair + 3]
    scr = refs[2 * npair + 3:]
    src_v, dst_v, bufs, accum = scr[:4]
    sems = scr[4:]

    c = lax.axis_index("c")
    s = lax.axis_index("s")
    wid = _worker(c, s)
    pltpu.sync_copy(src_hbm.at[wid], src_v)
    pltpu.sync_copy(dst_hbm.at[wid], dst_v)

    for y_hbm, out_hbm in zip(ys, outs):
        for b in range(NBUF):
            pltpu.async_copy(y_hbm.at[src_v.at[b]], bufs.at[b], sems[b])
        pltpu.sync_copy(zeros_hbm.at[pl.ds(s * RPT, RPT)],
                        accum.at[pl.ds(s * RPT, RPT)])
        plsc.subcore_barrier()

        @pl.loop(0, CH, step=NBUF)
        def _grp(i):
            for b in range(NBUF):
                ch = i + b
                pltpu.make_async_copy(y_hbm.at[src_v.at[ch]], bufs.at[b],
                                      sems[b]).wait()
                pltpu.sync_copy(bufs.at[b], accum.at[dst_v.at[ch]], add=True)

                @pl.when(ch + NBUF < CH)
                def _refill():
                    pltpu.async_copy(y_hbm.at[src_v.at[ch + NBUF]],
                                     bufs.at[b], sems[b])

        plsc.subcore_barrier()
        pltpu.sync_copy(accum.at[pl.ds(s * RPT, RPT)],
                        out_hbm.at[c, pl.ds(s * RPT, RPT)])
        plsc.subcore_barrier()


def _make_prop(fw, npair):
    return pl.kernel(
        functools.partial(_prop_body, fw, npair),
        out_type=[jax.ShapeDtypeStruct((NC, NA, fw), jnp.float32)] * npair,
        mesh=plsc.VectorSubcoreMesh(core_axis_name="c", subcore_axis_name="s",
                                    num_cores=NC, num_subcores=NS),
        compiler_params=pltpu.CompilerParams(use_tc_tiling_on_sc=False),
        scratch_types=[
            pltpu.VMEM((CH, K), jnp.int32),
            pltpu.VMEM((CH, K), jnp.int32),
            pltpu.VMEM((NBUF, K, fw), jnp.float32),
            pltpu.VMEM_SHARED((NA, fw), jnp.float32),
        ] + [pltpu.SemaphoreType.DMA] * NBUF,
    )


_prop128 = _make_prop(128, 1)
_prop48 = _make_prop(48, 1)


def _norm_from(degp_ref):
    deg = degp_ref[0, :, 0] + degp_ref[1, :, 0]
    return lax.rsqrt(jnp.maximum(deg, 1.0))


def _mm1_body(x_ref, w_ref, degp_ref, ya_ref, yb_ref):
    norm = _norm_from(degp_ref)
    y = jnp.dot(x_ref[...], w_ref[...],
                preferred_element_type=jnp.float32) * norm[:, None]
    ya_ref[...] = y[:, :128]
    yb_ref[...] = y[:, 128:]


_mm1 = pl.pallas_call(
    _mm1_body,
    grid=(N_NODES // BM,),
    in_specs=[
        pl.BlockSpec((BM, 256), lambda i: (i, 0)),
        pl.BlockSpec((256, 256), lambda i: (0, 0)),
        pl.BlockSpec((2, BM, 16), lambda i: (0, i, 0)),
    ],
    out_specs=[
        pl.BlockSpec((BM, 128), lambda i: (i, 0)),
        pl.BlockSpec((BM, 128), lambda i: (i, 0)),
    ],
    out_shape=[
        jax.ShapeDtypeStruct((N_NODES, 128), jnp.float32),
        jax.ShapeDtypeStruct((N_NODES, 128), jnp.float32),
    ],
)


def _mm2_body(pa_ref, pb_ref, degp_ref, b_ref, w_ref, ya_ref, yb_ref):
    norm = _norm_from(degp_ref)
    nc = norm[:, None]
    ha = jnp.maximum((pa_ref[0] + pa_ref[1]) * nc + b_ref[0, :128], 0.0)
    hb = jnp.maximum((pb_ref[0] + pb_ref[1]) * nc + b_ref[0, 128:], 0.0)
    y = (jnp.dot(ha, w_ref[:128, :], preferred_element_type=jnp.float32)
         + jnp.dot(hb, w_ref[128:, :], preferred_element_type=jnp.float32)) * nc
    ya_ref[...] = y[:, :128]
    yb_ref[...] = y[:, 128:]


_mm2 = pl.pallas_call(
    _mm2_body,
    grid=(N_NODES // BM,),
    in_specs=[
        pl.BlockSpec((2, BM, 128), lambda i: (0, i, 0)),
        pl.BlockSpec((2, BM, 128), lambda i: (0, i, 0)),
        pl.BlockSpec((2, BM, 16), lambda i: (0, i, 0)),
        pl.BlockSpec((1, 256), lambda i: (0, 0)),
        pl.BlockSpec((256, 256), lambda i: (0, 0)),
    ],
    out_specs=[
        pl.BlockSpec((BM, 128), lambda i: (i, 0)),
        pl.BlockSpec((BM, 128), lambda i: (i, 0)),
    ],
    out_shape=[
        jax.ShapeDtypeStruct((N_NODES, 128), jnp.float32),
        jax.ShapeDtypeStruct((N_NODES, 128), jnp.float32),
    ],
)


def _mm3_body(pa_ref, pb_ref, degp_ref, b_ref, w_ref, y_ref):
    norm = _norm_from(degp_ref)
    nc = norm[:, None]
    ha = jnp.maximum((pa_ref[0] + pa_ref[1]) * nc + b_ref[0, :128], 0.0)
    hb = jnp.maximum((pb_ref[0] + pb_ref[1]) * nc + b_ref[0, 128:], 0.0)
    y_ref[...] = (jnp.dot(ha, w_ref[:128, :], preferred_element_type=jnp.float32)
                  + jnp.dot(hb, w_ref[128:, :],
                            preferred_element_type=jnp.float32)) * nc


_mm3 = pl.pallas_call(
    _mm3_body,
    grid=(N_NODES // BM,),
    in_specs=[
        pl.BlockSpec((2, BM, 128), lambda i: (0, i, 0)),
        pl.BlockSpec((2, BM, 128), lambda i: (0, i, 0)),
        pl.BlockSpec((2, BM, 16), lambda i: (0, i, 0)),
        pl.BlockSpec((1, 256), lambda i: (0, 0)),
        pl.BlockSpec((256, 48), lambda i: (0, 0)),
    ],
    out_specs=pl.BlockSpec((BM, 48), lambda i: (i, 0)),
    out_shape=jax.ShapeDtypeStruct((N_NODES, 48), jnp.float32),
)


def _final_body(p_ref, degp_ref, b_ref, o_ref):
    norm = _norm_from(degp_ref)
    o_ref[...] = ((p_ref[0, :, :40] + p_ref[1, :, :40]) * norm[:, None]
                  + b_ref[0])


_final = pl.pallas_call(
    _final_body,
    grid=(N_NODES // BM,),
    in_specs=[
        pl.BlockSpec((2, BM, 48), lambda i: (0, i, 0)),
        pl.BlockSpec((2, BM, 16), lambda i: (0, i, 0)),
        pl.BlockSpec((1, 40), lambda i: (0, 0)),
    ],
    out_specs=pl.BlockSpec((BM, 40), lambda i: (i, 0)),
    out_shape=jax.ShapeDtypeStruct((N_NODES, 40), jnp.float32),
)


def kernel(features, edge_index, W1, b1, W2, b2, W3, b3):
    npad = EPT - EPW
    src = edge_index[0].reshape(NW, EPW)
    dst = edge_index[1].reshape(NW, EPW)
    pad_src = jnp.zeros((NW, npad), jnp.int32)
    pad_dst = jnp.broadcast_to(
        N_NODES + (jnp.arange(npad, dtype=jnp.int32) % (NA - N_NODES)),
        (NW, npad))
    src_t = jnp.concatenate([src, pad_src], axis=1).reshape(NW, CH, K)
    dst_t = jnp.concatenate([dst, pad_dst], axis=1).reshape(NW, CH, K)

    ones16 = jnp.ones((K, 16), jnp.float32)
    zeros16 = jnp.zeros((NA, 16), jnp.float32)
    zeros128 = jnp.zeros((NA, 128), jnp.float32)
    zeros48 = jnp.zeros((NA, 48), jnp.float32)
    W3p = jnp.pad(W3, ((0, 0), (0, 48 - W3.shape[1])))
    b1r = b1.reshape(1, 256)
    b2r = b2.reshape(1, 256)
    b3r = b3.reshape(1, 40)

    degp = _deg_kernel(dst_t, ones16, zeros16)
    y1a, y1b = _mm1(features, W1, degp)
    (p1a,) = _prop128(y1a, src_t, dst_t, zeros128)
    (p1b,) = _prop128(y1b, src_t, dst_t, zeros128)
    y2a, y2b = _mm2(p1a, p1b, degp, b1r, W2)
    (p2a,) = _prop128(y2a, src_t, dst_t, zeros128)
    (p2b,) = _prop128(y2b, src_t, dst_t, zeros128)
    y3 = _mm3(p2a, p2b, degp, b2r, W3p)
    (p3,) = _prop48(y3, src_t, dst_t, zeros48)
    return _final(p3, degp, b3r)


# speedup vs baseline: 3.4693x; 3.4693x over previous
import functools

import jax
import jax.numpy as jnp
from jax import lax
from jax.experimental import pallas as pl
from jax.experimental.pallas import tpu as pltpu
from jax.experimental.pallas import tpu_sc as plsc

N_NODES = 10000
N_EDGES = 160000
NC, NS = 2, 16
NW = NC * NS
K = 40
CH = 125
EPT = CH * K
EPW = N_EDGES // NW
NBUF = 5
NA = 10112
RPT = NA // NS

BM = 1000


def _worker(c, s):
    return s * NC + c


def _deg_body(dst_hbm, ones_hbm, zeros_hbm, out_hbm, idx_v, ones_v, accum, sem):
    c = lax.axis_index("c")
    s = lax.axis_index("s")
    wid = _worker(c, s)
    pltpu.sync_copy(dst_hbm.at[wid], idx_v)
    pltpu.sync_copy(ones_hbm, ones_v)
    pltpu.sync_copy(zeros_hbm.at[pl.ds(s * RPT, RPT)],
                    accum.at[pl.ds(s * RPT, RPT)])
    plsc.subcore_barrier()

    @pl.loop(0, CH, step=5)
    def _grp(i):
        for b in range(5):
            pltpu.async_copy(ones_v, accum.at[idx_v.at[i + b]], sem, add=True)
        for b in range(5):
            pltpu.make_async_copy(ones_v, accum.at[idx_v.at[i + b]], sem).wait()

    plsc.subcore_barrier()
    pltpu.sync_copy(accum.at[pl.ds(s * RPT, RPT)],
                    out_hbm.at[c, pl.ds(s * RPT, RPT)])


_deg_kernel = pl.kernel(
    _deg_body,
    out_type=jax.ShapeDtypeStruct((NC, NA, 16), jnp.float32),
    mesh=plsc.VectorSubcoreMesh(core_axis_name="c", subcore_axis_name="s",
                                num_cores=NC, num_subcores=NS),
    compiler_params=pltpu.CompilerParams(use_tc_tiling_on_sc=False),
    scratch_types=[
        pltpu.VMEM((CH, K), jnp.int32),
        pltpu.VMEM((K, 16), jnp.float32),
        pltpu.VMEM_SHARED((NA, 16), jnp.float32),
        pltpu.SemaphoreType.DMA,
    ],
)


def _prop_body(fw, npair, *refs):
    ys = refs[:npair]
    src_hbm, dst_hbm, zeros_hbm = refs[npair:npair + 3]
    outs = refs[npair + 3:2 * npair + 3]
    scr = refs[2 * npair + 3:]
    src_v, dst_v, bufs, accum = scr[:4]
    sems = scr[4:]

    c = lax.axis_index("c")
    s = lax.axis_index("s")
    wid = _worker(c, s)
    pltpu.sync_copy(src_hbm.at[wid], src_v)
    pltpu.sync_copy(dst_hbm.at[wid], dst_v)

    for y_hbm, out_hbm in zip(ys, outs):
        for b in range(NBUF):
            pltpu.async_copy(y_hbm.at[src_v.at[b]], bufs.at[b], sems[b])
        pltpu.sync_copy(zeros_hbm.at[pl.ds(s * RPT, RPT)],
                        accum.at[pl.ds(s * RPT, RPT)])
        plsc.subcore_barrier()

        @pl.loop(0, CH, step=NBUF)
        def _grp(i):
            for b in range(NBUF):
                ch = i + b
                pltpu.make_async_copy(y_hbm.at[src_v.at[ch]], bufs.at[b],
                                      sems[b]).wait()
                pltpu.sync_copy(bufs.at[b], accum.at[dst_v.at[ch]], add=True)

                @pl.when(ch + NBUF < CH)
                def _refill():
                    pltpu.async_copy(y_hbm.at[src_v.at[ch + NBUF]],
                                     bufs.at[b], sems[b])

        plsc.subcore_barrier()
        pltpu.sync_copy(accum.at[pl.ds(s * RPT, RPT)],
                        out_hbm.at[c, pl.ds(s * RPT, RPT)])
        plsc.subcore_barrier()


def _make_prop(fw, npair):
    return pl.kernel(
        functools.partial(_prop_body, fw, npair),
        out_type=[jax.ShapeDtypeStruct((NC, NA, fw), jnp.float32)] * npair,
        mesh=plsc.VectorSubcoreMesh(core_axis_name="c", subcore_axis_name="s",
                                    num_cores=NC, num_subcores=NS),
        compiler_params=pltpu.CompilerParams(use_tc_tiling_on_sc=False),
        scratch_types=[
            pltpu.VMEM((CH, K), jnp.int32),
            pltpu.VMEM((CH, K), jnp.int32),
            pltpu.VMEM((NBUF, K, fw), jnp.float32),
            pltpu.VMEM_SHARED((NA, fw), jnp.float32),
        ] + [pltpu.SemaphoreType.DMA] * NBUF,
    )


_prop128 = _make_prop(128, 1)
_prop48 = _make_prop(48, 1)


def _norm_from(degp_ref):
    deg = degp_ref[0, :, 0] + degp_ref[1, :, 0]
    return lax.rsqrt(jnp.maximum(deg, 1.0))


def _mm1_body(x_ref, w_ref, degp_ref, ya_ref, yb_ref):
    norm = _norm_from(degp_ref)
    y = jnp.dot(x_ref[...], w_ref[...],
                preferred_element_type=jnp.float32) * norm[:, None]
    ya_ref[...] = y[:, :128]
    yb_ref[...] = y[:, 128:]


_mm1 = pl.pallas_call(
    _mm1_body,
    grid=(N_NODES // BM,),
    in_specs=[
        pl.BlockSpec((BM, 256), lambda i: (i, 0)),
        pl.BlockSpec((256, 256), lambda i: (0, 0)),
        pl.BlockSpec((2, BM, 16), lambda i: (0, i, 0)),
    ],
    out_specs=[
        pl.BlockSpec((BM, 128), lambda i: (i, 0)),
        pl.BlockSpec((BM, 128), lambda i: (i, 0)),
    ],
    out_shape=[
        jax.ShapeDtypeStruct((N_NODES, 128), jnp.float32),
        jax.ShapeDtypeStruct((N_NODES, 128), jnp.float32),
    ],
)


def _mm2_body(pa_ref, pb_ref, degp_ref, b_ref, w_ref, ya_ref, yb_ref):
    norm = _norm_from(degp_ref)
    nc = norm[:, None]
    ha = jnp.maximum((pa_ref[0] + pa_ref[1]) * nc + b_ref[0, :128], 0.0)
    hb = jnp.maximum((pb_ref[0] + pb_ref[1]) * nc + b_ref[0, 128:], 0.0)
    y = (jnp.dot(ha, w_ref[:128, :], preferred_element_type=jnp.float32)
         + jnp.dot(hb, w_ref[128:, :], preferred_element_type=jnp.float32)) * nc
    ya_ref[...] = y[:, :128]
    yb_ref[...] = y[:, 128:]


_mm2 = pl.pallas_call(
    _mm2_body,
    grid=(N_NODES // BM,),
    in_specs=[
        pl.BlockSpec((2, BM, 128), lambda i: (0, i, 0)),
        pl.BlockSpec((2, BM, 128), lambda i: (0, i, 0)),
        pl.BlockSpec((2, BM, 16), lambda i: (0, i, 0)),
        pl.BlockSpec((1, 256), lambda i: (0, 0)),
        pl.BlockSpec((256, 256), lambda i: (0, 0)),
    ],
    out_specs=[
        pl.BlockSpec((BM, 128), lambda i: (i, 0)),
        pl.BlockSpec((BM, 128), lambda i: (i, 0)),
    ],
    out_shape=[
        jax.ShapeDtypeStruct((N_NODES, 128), jnp.float32),
        jax.ShapeDtypeStruct((N_NODES, 128), jnp.float32),
    ],
)


def _mm3_body(pa_ref, pb_ref, degp_ref, b_ref, w_ref, y_ref):
    norm = _norm_from(degp_ref)
    nc = norm[:, None]
    ha = jnp.maximum((pa_ref[0] + pa_ref[1]) * nc + b_ref[0, :128], 0.0)
    hb = jnp.maximum((pb_ref[0] + pb_ref[1]) * nc + b_ref[0, 128:], 0.0)
    y_ref[...] = (jnp.dot(ha, w_ref[:128, :], preferred_element_type=jnp.float32)
                  + jnp.dot(hb, w_ref[128:, :],
                            preferred_element_type=jnp.float32)) * nc


_mm3 = pl.pallas_call(
    _mm3_body,
    grid=(N_NODES // BM,),
    in_specs=[
        pl.BlockSpec((2, BM, 128), lambda i: (0, i, 0)),
        pl.BlockSpec((2, BM, 128), lambda i: (0, i, 0)),
        pl.BlockSpec((2, BM, 16), lambda i: (0, i, 0)),
        pl.BlockSpec((1, 256), lambda i: (0, 0)),
        pl.BlockSpec((256, 48), lambda i: (0, 0)),
    ],
    out_specs=pl.BlockSpec((BM, 48), lambda i: (i, 0)),
    out_shape=jax.ShapeDtypeStruct((N_NODES, 48), jnp.float32),
)


def _final_body(p_ref, degp_ref, b_ref, o_ref):
    norm = _norm_from(degp_ref)
    o_ref[...] = ((p_ref[0, :, :40] + p_ref[1, :, :40]) * norm[:, None]
                  + b_ref[0])


_final = pl.pallas_call(
    _final_body,
    grid=(N_NODES // BM,),
    in_specs=[
        pl.BlockSpec((2, BM, 48), lambda i: (0, i, 0)),
        pl.BlockSpec((2, BM, 16), lambda i: (0, i, 0)),
        pl.BlockSpec((1, 40), lambda i: (0, 0)),
    ],
    out_specs=pl.BlockSpec((BM, 40), lambda i: (i, 0)),
    out_shape=jax.ShapeDtypeStruct((N_NODES, 40), jnp.float32),
)


def kernel(features, edge_index, W1, b1, W2, b2, W3, b3):
    src_t = edge_index[0].reshape(NW, CH, K)
    dst_t = edge_index[1].reshape(NW, CH, K)

    ones16 = jnp.ones((K, 16), jnp.float32)
    zeros16 = jnp.zeros((NA, 16), jnp.float32)
    zeros128 = jnp.zeros((NA, 128), jnp.float32)
    zeros48 = jnp.zeros((NA, 48), jnp.float32)
    W3p = jnp.pad(W3, ((0, 0), (0, 48 - W3.shape[1])))
    b1r = b1.reshape(1, 256)
    b2r = b2.reshape(1, 256)
    b3r = b3.reshape(1, 40)

    degp = _deg_kernel(dst_t, ones16, zeros16)
    y1a, y1b = _mm1(features, W1, degp)
    (p1a,) = _prop128(y1a, src_t, dst_t, zeros128)
    (p1b,) = _prop128(y1b, src_t, dst_t, zeros128)
    y2a, y2b = _mm2(p1a, p1b, degp, b1r, W2)
    (p2a,) = _prop128(y2a, src_t, dst_t, zeros128)
    (p2b,) = _prop128(y2b, src_t, dst_t, zeros128)
    y3 = _mm3(p2a, p2b, degp, b2r, W3p)
    (p3,) = _prop48(y3, src_t, dst_t, zeros48)
    return _final(p3, degp, b3r)


# speedup vs baseline: 3.5385x; 1.0199x over previous
import functools

import jax
import jax.numpy as jnp
from jax import lax
from jax.experimental import pallas as pl
from jax.experimental.pallas import tpu as pltpu
from jax.experimental.pallas import tpu_sc as plsc

N_NODES = 10000
N_EDGES = 160000
NC, NS = 2, 16
NW = NC * NS
K = 40
CH = 125
EPT = CH * K
EPW = N_EDGES // NW
NBUF = 5
NA = 10112
RPT = NA // NS

BM = 1000


def _worker(c, s):
    return s * NC + c


def _deg_body(dst_hbm, ones_hbm, zeros_hbm, out_hbm, idx_v, ones_v, accum, sem):
    c = lax.axis_index("c")
    s = lax.axis_index("s")
    wid = _worker(c, s)
    pltpu.sync_copy(dst_hbm.at[wid], idx_v)
    pltpu.sync_copy(ones_hbm, ones_v)
    pltpu.sync_copy(zeros_hbm.at[pl.ds(s * RPT, RPT)],
                    accum.at[pl.ds(s * RPT, RPT)])
    plsc.subcore_barrier()

    @pl.loop(0, CH, step=5)
    def _grp(i):
        for b in range(5):
            pltpu.async_copy(ones_v, accum.at[idx_v.at[i + b]], sem, add=True)
        for b in range(5):
            pltpu.make_async_copy(ones_v, accum.at[idx_v.at[i + b]], sem).wait()

    plsc.subcore_barrier()
    pltpu.sync_copy(accum.at[pl.ds(s * RPT, RPT)],
                    out_hbm.at[c, pl.ds(s * RPT, RPT)])


_deg_kernel = pl.kernel(
    _deg_body,
    out_type=jax.ShapeDtypeStruct((NC, NA, 16), jnp.float32),
    mesh=plsc.VectorSubcoreMesh(core_axis_name="c", subcore_axis_name="s",
                                num_cores=NC, num_subcores=NS),
    compiler_params=pltpu.CompilerParams(use_tc_tiling_on_sc=False),
    scratch_types=[
        pltpu.VMEM((CH, K), jnp.int32),
        pltpu.VMEM((K, 16), jnp.float32),
        pltpu.VMEM_SHARED((NA, 16), jnp.float32),
        pltpu.SemaphoreType.DMA,
    ],
)


def _prop_body(fw, npair, *refs):
    ys = refs[:npair]
    src_hbm, dst_hbm, zeros_hbm = refs[npair:npair + 3]
    outs = refs[npair + 3:2 * npair + 3]
    scr = refs[2 * npair + 3:]
    src_v, dst_v, bufs, accum = scr[:4]
    sems = scr[4:]

    c = lax.axis_index("c")
    s = lax.axis_index("s")
    wid = _worker(c, s)
    pltpu.sync_copy(src_hbm.at[wid], src_v)
    pltpu.sync_copy(dst_hbm.at[wid], dst_v)

    for y_hbm, out_hbm in zip(ys, outs):
        for b in range(NBUF):
            pltpu.async_copy(y_hbm.at[src_v.at[b]], bufs.at[b], sems[b])
        pltpu.sync_copy(zeros_hbm.at[pl.ds(s * RPT, RPT)],
                        accum.at[pl.ds(s * RPT, RPT)])
        plsc.subcore_barrier()

        @pl.loop(0, CH, step=NBUF)
        def _grp(i):
            for b in range(NBUF):
                ch = i + b
                pltpu.make_async_copy(y_hbm.at[src_v.at[ch]], bufs.at[b],
                                      sems[b]).wait()
                pltpu.sync_copy(bufs.at[b], accum.at[dst_v.at[ch]], add=True)

                @pl.when(ch + NBUF < CH)
                def _refill():
                    pltpu.async_copy(y_hbm.at[src_v.at[ch + NBUF]],
                                     bufs.at[b], sems[b])

        plsc.subcore_barrier()
        pltpu.sync_copy(accum.at[pl.ds(s * RPT, RPT)],
                        out_hbm.at[c, pl.ds(s * RPT, RPT)])
        plsc.subcore_barrier()


def _make_prop(fw, npair):
    return pl.kernel(
        functools.partial(_prop_body, fw, npair),
        out_type=[jax.ShapeDtypeStruct((NC, NA, fw), jnp.float32)] * npair,
        mesh=plsc.VectorSubcoreMesh(core_axis_name="c", subcore_axis_name="s",
                                    num_cores=NC, num_subcores=NS),
        compiler_params=pltpu.CompilerParams(use_tc_tiling_on_sc=False),
        scratch_types=[
            pltpu.VMEM((CH, K), jnp.int32),
            pltpu.VMEM((CH, K), jnp.int32),
            pltpu.VMEM((NBUF, K, fw), jnp.float32),
            pltpu.VMEM_SHARED((NA, fw), jnp.float32),
        ] + [pltpu.SemaphoreType.DMA] * NBUF,
    )


_prop128 = _make_prop(128, 2)
_prop48 = _make_prop(48, 1)


def _norm_from(degp_ref):
    deg = degp_ref[0, :, 0] + degp_ref[1, :, 0]
    return lax.rsqrt(jnp.maximum(deg, 1.0))


def _mm1_body(x_ref, w_ref, degp_ref, ya_ref, yb_ref):
    norm = _norm_from(degp_ref)
    y = jnp.dot(x_ref[...], w_ref[...],
                preferred_element_type=jnp.float32) * norm[:, None]
    ya_ref[...] = y[:, :128]
    yb_ref[...] = y[:, 128:]


_mm1 = pl.pallas_call(
    _mm1_body,
    grid=(N_NODES // BM,),
    in_specs=[
        pl.BlockSpec((BM, 256), lambda i: (i, 0)),
        pl.BlockSpec((256, 256), lambda i: (0, 0)),
        pl.BlockSpec((2, BM, 16), lambda i: (0, i, 0)),
    ],
    out_specs=[
        pl.BlockSpec((BM, 128), lambda i: (i, 0)),
        pl.BlockSpec((BM, 128), lambda i: (i, 0)),
    ],
    out_shape=[
        jax.ShapeDtypeStruct((N_NODES, 128), jnp.float32),
        jax.ShapeDtypeStruct((N_NODES, 128), jnp.float32),
    ],
)


def _mm2_body(pa_ref, pb_ref, degp_ref, b_ref, w_ref, ya_ref, yb_ref):
    norm = _norm_from(degp_ref)
    nc = norm[:, None]
    ha = jnp.maximum((pa_ref[0] + pa_ref[1]) * nc + b_ref[0, :128], 0.0)
    hb = jnp.maximum((pb_ref[0] + pb_ref[1]) * nc + b_ref[0, 128:], 0.0)
    y = (jnp.dot(ha, w_ref[:128, :], preferred_element_type=jnp.float32)
         + jnp.dot(hb, w_ref[128:, :], preferred_element_type=jnp.float32)) * nc
    ya_ref[...] = y[:, :128]
    yb_ref[...] = y[:, 128:]


_mm2 = pl.pallas_call(
    _mm2_body,
    grid=(N_NODES // BM,),
    in_specs=[
        pl.BlockSpec((2, BM, 128), lambda i: (0, i, 0)),
        pl.BlockSpec((2, BM, 128), lambda i: (0, i, 0)),
        pl.BlockSpec((2, BM, 16), lambda i: (0, i, 0)),
        pl.BlockSpec((1, 256), lambda i: (0, 0)),
        pl.BlockSpec((256, 256), lambda i: (0, 0)),
    ],
    out_specs=[
        pl.BlockSpec((BM, 128), lambda i: (i, 0)),
        pl.BlockSpec((BM, 128), lambda i: (i, 0)),
    ],
    out_shape=[
        jax.ShapeDtypeStruct((N_NODES, 128), jnp.float32),
        jax.ShapeDtypeStruct((N_NODES, 128), jnp.float32),
    ],
)


def _mm3_body(pa_ref, pb_ref, degp_ref, b_ref, w_ref, y_ref):
    norm = _norm_from(degp_ref)
    nc = norm[:, None]
    ha = jnp.maximum((pa_ref[0] + pa_ref[1]) * nc + b_ref[0, :128], 0.0)
    hb = jnp.maximum((pb_ref[0] + pb_ref[1]) * nc + b_ref[0, 128:], 0.0)
    y_ref[...] = (jnp.dot(ha, w_ref[:128, :], preferred_element_type=jnp.float32)
                  + jnp.dot(hb, w_ref[128:, :],
                            preferred_element_type=jnp.float32)) * nc


_mm3 = pl.pallas_call(
    _mm3_body,
    grid=(N_NODES // BM,),
    in_specs=[
        pl.BlockSpec((2, BM, 128), lambda i: (0, i, 0)),
        pl.BlockSpec((2, BM, 128), lambda i: (0, i, 0)),
        pl.BlockSpec((2, BM, 16), lambda i: (0, i, 0)),
        pl.BlockSpec((1, 256), lambda i: (0, 0)),
        pl.BlockSpec((256, 48), lambda i: (0, 0)),
    ],
    out_specs=pl.BlockSpec((BM, 48), lambda i: (i, 0)),
    out_shape=jax.ShapeDtypeStruct((N_NODES, 48), jnp.float32),
)


def _final_body(p_ref, degp_ref, b_ref, o_ref):
    norm = _norm_from(degp_ref)
    o_ref[...] = ((p_ref[0, :, :40] + p_ref[1, :, :40]) * norm[:, None]
                  + b_ref[0])


_final = pl.pallas_call(
    _final_body,
    grid=(N_NODES // BM,),
    in_specs=[
        pl.BlockSpec((2, BM, 48), lambda i: (0, i, 0)),
        pl.BlockSpec((2, BM, 16), lambda i: (0, i, 0)),
        pl.BlockSpec((1, 40), lambda i: (0, 0)),
    ],
    out_specs=pl.BlockSpec((BM, 40), lambda i: (i, 0)),
    out_shape=jax.ShapeDtypeStruct((N_NODES, 40), jnp.float32),
)


def kernel(features, edge_index, W1, b1, W2, b2, W3, b3):
    src_t = edge_index[0].reshape(NW, CH, K)
    dst_t = edge_index[1].reshape(NW, CH, K)

    ones16 = jnp.ones((K, 16), jnp.float32)
    zeros16 = jnp.zeros((NA, 16), jnp.float32)
    zeros128 = jnp.zeros((NA, 128), jnp.float32)
    zeros48 = jnp.zeros((NA, 48), jnp.float32)
    W3p = jnp.pad(W3, ((0, 0), (0, 48 - W3.shape[1])))
    b1r = b1.reshape(1, 256)
    b2r = b2.reshape(1, 256)
    b3r = b3.reshape(1, 40)

    degp = _deg_kernel(dst_t, ones16, zeros16)
    y1a, y1b = _mm1(features, W1, degp)
    p1a, p1b = _prop128(y1a, y1b, src_t, dst_t, zeros128)
    y2a, y2b = _mm2(p1a, p1b, degp, b1r, W2)
    p2a, p2b = _prop128(y2a, y2b, src_t, dst_t, zeros128)
    y3 = _mm3(p2a, p2b, degp, b2r, W3p)
    (p3,) = _prop48(y3, src_t, dst_t, zeros48)
    return _final(p3, degp, b3r)


# speedup vs baseline: 3.6256x; 1.0246x over previous
import functools

import jax
import jax.numpy as jnp
from jax import lax
from jax.experimental import pallas as pl
from jax.experimental.pallas import tpu as pltpu
from jax.experimental.pallas import tpu_sc as plsc

N_NODES = 10000
N_EDGES = 160000
NC, NS = 2, 16
NW = NC * NS
K = 40
CH = 125
EPT = CH * K
EPW = N_EDGES // NW
NBUF = 5
NA = 10112
RPT = NA // NS

BM = 1000


def _worker(c, s):
    return s * NC + c


def _deg_body(dst_hbm, ones_hbm, zeros_hbm, out_hbm, idx_v, ones_v, accum, sem):
    c = lax.axis_index("c")
    s = lax.axis_index("s")
    wid = _worker(c, s)
    pltpu.sync_copy(dst_hbm.at[wid], idx_v)
    pltpu.sync_copy(ones_hbm, ones_v)
    pltpu.sync_copy(zeros_hbm.at[pl.ds(s * RPT, RPT), pl.ds(0, 16)],
                    accum.at[pl.ds(s * RPT, RPT)])
    plsc.subcore_barrier()

    @pl.loop(0, CH, step=5)
    def _grp(i):
        for b in range(5):
            pltpu.async_copy(ones_v, accum.at[idx_v.at[i + b]], sem, add=True)
        for b in range(5):
            pltpu.make_async_copy(ones_v, accum.at[idx_v.at[i + b]], sem).wait()

    plsc.subcore_barrier()
    pltpu.sync_copy(accum.at[pl.ds(s * RPT, RPT)],
                    out_hbm.at[c, pl.ds(s * RPT, RPT)])


_deg_kernel = pl.kernel(
    _deg_body,
    out_type=jax.ShapeDtypeStruct((NC, NA, 16), jnp.float32),
    mesh=plsc.VectorSubcoreMesh(core_axis_name="c", subcore_axis_name="s",
                                num_cores=NC, num_subcores=NS),
    compiler_params=pltpu.CompilerParams(use_tc_tiling_on_sc=False),
    scratch_types=[
        pltpu.VMEM((CH, K), jnp.int32),
        pltpu.VMEM((K, 16), jnp.float32),
        pltpu.VMEM_SHARED((NA, 16), jnp.float32),
        pltpu.SemaphoreType.DMA,
    ],
)


def _prop_body(fw, npair, *refs):
    ys = refs[:npair]
    src_hbm, dst_hbm, zeros_hbm = refs[npair:npair + 3]
    outs = refs[npair + 3:2 * npair + 3]
    scr = refs[2 * npair + 3:]
    src_v, dst_v, bufs, accum = scr[:4]
    sems = scr[4:]

    c = lax.axis_index("c")
    s = lax.axis_index("s")
    wid = _worker(c, s)
    pltpu.sync_copy(src_hbm.at[wid], src_v)
    pltpu.sync_copy(dst_hbm.at[wid], dst_v)

    for b in range(NBUF):
        pltpu.async_copy(ys[0].at[src_v.at[b]], bufs.at[b], sems[b])

    for h, (y_hbm, out_hbm) in enumerate(zip(ys, outs)):
        pltpu.sync_copy(zeros_hbm.at[pl.ds(s * RPT, RPT), pl.ds(0, fw)],
                        accum.at[pl.ds(s * RPT, RPT)])
        plsc.subcore_barrier()

        @pl.loop(0, CH, step=NBUF)
        def _grp(i):
            for b in range(NBUF):
                ch = i + b
                pltpu.make_async_copy(y_hbm.at[src_v.at[ch]], bufs.at[b],
                                      sems[b]).wait()
                pltpu.sync_copy(bufs.at[b], accum.at[dst_v.at[ch]], add=True)

                @pl.when(ch + NBUF < CH)
                def _refill():
                    pltpu.async_copy(y_hbm.at[src_v.at[ch + NBUF]],
                                     bufs.at[b], sems[b])

        plsc.subcore_barrier()
        if h + 1 < len(ys):
            for b in range(NBUF):
                pltpu.async_copy(ys[h + 1].at[src_v.at[b]], bufs.at[b],
                                 sems[b])
        pltpu.sync_copy(accum.at[pl.ds(s * RPT, RPT)],
                        out_hbm.at[c, pl.ds(s * RPT, RPT)])


def _make_prop(fw, npair):
    return pl.kernel(
        functools.partial(_prop_body, fw, npair),
        out_type=[jax.ShapeDtypeStruct((NC, NA, fw), jnp.float32)] * npair,
        mesh=plsc.VectorSubcoreMesh(core_axis_name="c", subcore_axis_name="s",
                                    num_cores=NC, num_subcores=NS),
        compiler_params=pltpu.CompilerParams(use_tc_tiling_on_sc=False),
        scratch_types=[
            pltpu.VMEM((CH, K), jnp.int32),
            pltpu.VMEM((CH, K), jnp.int32),
            pltpu.VMEM((NBUF, K, fw), jnp.float32),
            pltpu.VMEM_SHARED((NA, fw), jnp.float32),
        ] + [pltpu.SemaphoreType.DMA] * NBUF,
    )


_prop128 = _make_prop(128, 2)
_prop48 = _make_prop(48, 1)


def _norm_from(degp_ref):
    deg = degp_ref[0, :, 0] + degp_ref[1, :, 0]
    return lax.rsqrt(jnp.maximum(deg, 1.0))


def _mm1_body(x_ref, w_ref, degp_ref, ya_ref, yb_ref):
    norm = _norm_from(degp_ref)
    y = jnp.dot(x_ref[...], w_ref[...],
                preferred_element_type=jnp.float32) * norm[:, None]
    ya_ref[...] = y[:, :128]
    yb_ref[...] = y[:, 128:]


_mm1 = pl.pallas_call(
    _mm1_body,
    grid=(N_NODES // BM,),
    in_specs=[
        pl.BlockSpec((BM, 256), lambda i: (i, 0)),
        pl.BlockSpec((256, 256), lambda i: (0, 0)),
        pl.BlockSpec((2, BM, 16), lambda i: (0, i, 0)),
    ],
    out_specs=[
        pl.BlockSpec((BM, 128), lambda i: (i, 0)),
        pl.BlockSpec((BM, 128), lambda i: (i, 0)),
    ],
    out_shape=[
        jax.ShapeDtypeStruct((N_NODES, 128), jnp.float32),
        jax.ShapeDtypeStruct((N_NODES, 128), jnp.float32),
    ],
)


def _mm2_body(pa_ref, pb_ref, degp_ref, b_ref, w_ref, ya_ref, yb_ref):
    norm = _norm_from(degp_ref)
    nc = norm[:, None]
    ha = jnp.maximum((pa_ref[0] + pa_ref[1]) * nc + b_ref[0, :128], 0.0)
    hb = jnp.maximum((pb_ref[0] + pb_ref[1]) * nc + b_ref[0, 128:], 0.0)
    y = (jnp.dot(ha, w_ref[:128, :], preferred_element_type=jnp.float32)
         + jnp.dot(hb, w_ref[128:, :], preferred_element_type=jnp.float32)) * nc
    ya_ref[...] = y[:, :128]
    yb_ref[...] = y[:, 128:]


_mm2 = pl.pallas_call(
    _mm2_body,
    grid=(N_NODES // BM,),
    in_specs=[
        pl.BlockSpec((2, BM, 128), lambda i: (0, i, 0)),
        pl.BlockSpec((2, BM, 128), lambda i: (0, i, 0)),
        pl.BlockSpec((2, BM, 16), lambda i: (0, i, 0)),
        pl.BlockSpec((1, 256), lambda i: (0, 0)),
        pl.BlockSpec((256, 256), lambda i: (0, 0)),
    ],
    out_specs=[
        pl.BlockSpec((BM, 128), lambda i: (i, 0)),
        pl.BlockSpec((BM, 128), lambda i: (i, 0)),
    ],
    out_shape=[
        jax.ShapeDtypeStruct((N_NODES, 128), jnp.float32),
        jax.ShapeDtypeStruct((N_NODES, 128), jnp.float32),
    ],
)


def _mm3_body(pa_ref, pb_ref, degp_ref, b_ref, w_ref, y_ref):
    norm = _norm_from(degp_ref)
    nc = norm[:, None]
    ha = jnp.maximum((pa_ref[0] + pa_ref[1]) * nc + b_ref[0, :128], 0.0)
    hb = jnp.maximum((pb_ref[0] + pb_ref[1]) * nc + b_ref[0, 128:], 0.0)
    y_ref[...] = (jnp.dot(ha, w_ref[:128, :], preferred_element_type=jnp.float32)
                  + jnp.dot(hb, w_ref[128:, :],
                            preferred_element_type=jnp.float32)) * nc


_mm3 = pl.pallas_call(
    _mm3_body,
    grid=(N_NODES // BM,),
    in_specs=[
        pl.BlockSpec((2, BM, 128), lambda i: (0, i, 0)),
        pl.BlockSpec((2, BM, 128), lambda i: (0, i, 0)),
        pl.BlockSpec((2, BM, 16), lambda i: (0, i, 0)),
        pl.BlockSpec((1, 256), lambda i: (0, 0)),
        pl.BlockSpec((256, 48), lambda i: (0, 0)),
    ],
    out_specs=pl.BlockSpec((BM, 48), lambda i: (i, 0)),
    out_shape=jax.ShapeDtypeStruct((N_NODES, 48), jnp.float32),
)


def _final_body(p_ref, degp_ref, b_ref, o_ref):
    norm = _norm_from(degp_ref)
    o_ref[...] = ((p_ref[0, :, :40] + p_ref[1, :, :40]) * norm[:, None]
                  + b_ref[0])


_final = pl.pallas_call(
    _final_body,
    grid=(N_NODES // BM,),
    in_specs=[
        pl.BlockSpec((2, BM, 48), lambda i: (0, i, 0)),
        pl.BlockSpec((2, BM, 16), lambda i: (0, i, 0)),
        pl.BlockSpec((1, 40), lambda i: (0, 0)),
    ],
    out_specs=pl.BlockSpec((BM, 40), lambda i: (i, 0)),
    out_shape=jax.ShapeDtypeStruct((N_NODES, 40), jnp.float32),
)


def kernel(features, edge_index, W1, b1, W2, b2, W3, b3):
    src_t = edge_index[0].reshape(NW, CH, K)
    dst_t = edge_index[1].reshape(NW, CH, K)

    ones16 = jnp.ones((K, 16), jnp.float32)
    zeros128 = jnp.zeros((NA, 128), jnp.float32)
    W3p = jnp.pad(W3, ((0, 0), (0, 48 - W3.shape[1])))
    b1r = b1.reshape(1, 256)
    b2r = b2.reshape(1, 256)
    b3r = b3.reshape(1, 40)

    degp = _deg_kernel(dst_t, ones16, zeros128)
    y1a, y1b = _mm1(features, W1, degp)
    p1a, p1b = _prop128(y1a, y1b, src_t, dst_t, zeros128)
    y2a, y2b = _mm2(p1a, p1b, degp, b1r, W2)
    p2a, p2b = _prop128(y2a, y2b, src_t, dst_t, zeros128)
    y3 = _mm3(p2a, p2b, degp, b2r, W3p)
    (p3,) = _prop48(y3, src_t, dst_t, zeros128)
    return _final(p3, degp, b3r)


# speedup vs baseline: 3.8442x; 1.0603x over previous
import functools

import jax
import jax.numpy as jnp
from jax import lax
from jax.experimental import pallas as pl
from jax.experimental.pallas import tpu as pltpu
from jax.experimental.pallas import tpu_sc as plsc

N_NODES = 10000
N_EDGES = 160000
NC, NS = 2, 16
NW = NC * NS
K = 40
CH = 125
EPT = CH * K
EPW = N_EDGES // NW
NBUF = 5
NA = 10112
RPT = NA // NS

CH2 = N_EDGES // (NS * K)

BM = 1000


def _worker(c, s):
    return s * NC + c


def _deg_body(dst_hbm, ones_hbm, zeros_hbm, out_hbm, idx_v, ones_v, accum, sem):
    c = lax.axis_index("c")
    s = lax.axis_index("s")
    wid = _worker(c, s)
    pltpu.sync_copy(dst_hbm.at[wid], idx_v)
    pltpu.sync_copy(ones_hbm, ones_v)
    pltpu.sync_copy(zeros_hbm.at[pl.ds(s * RPT, RPT), pl.ds(0, 16)],
                    accum.at[pl.ds(s * RPT, RPT)])
    plsc.subcore_barrier()

    @pl.loop(0, CH, step=5)
    def _grp(i):
        for b in range(5):
            pltpu.async_copy(ones_v, accum.at[idx_v.at[i + b]], sem, add=True)
        for b in range(5):
            pltpu.make_async_copy(ones_v, accum.at[idx_v.at[i + b]], sem).wait()

    plsc.subcore_barrier()
    pltpu.sync_copy(accum.at[pl.ds(s * RPT, RPT)],
                    out_hbm.at[c, pl.ds(s * RPT, RPT)])


_deg_kernel = pl.kernel(
    _deg_body,
    out_type=jax.ShapeDtypeStruct((NC, NA, 16), jnp.float32),
    mesh=plsc.VectorSubcoreMesh(core_axis_name="c", subcore_axis_name="s",
                                num_cores=NC, num_subcores=NS),
    compiler_params=pltpu.CompilerParams(use_tc_tiling_on_sc=False),
    scratch_types=[
        pltpu.VMEM((CH, K), jnp.int32),
        pltpu.VMEM((K, 16), jnp.float32),
        pltpu.VMEM_SHARED((NA, 16), jnp.float32),
        pltpu.SemaphoreType.DMA,
    ],
)


def _prop_body(fw, npair, *refs):
    ys = refs[:npair]
    src_hbm, dst_hbm, zeros_hbm = refs[npair:npair + 3]
    outs = refs[npair + 3:2 * npair + 3]
    scr = refs[2 * npair + 3:]
    src_v, dst_v, bufs, accum = scr[:4]
    sems = scr[4:]

    c = lax.axis_index("c")
    s = lax.axis_index("s")
    wid = _worker(c, s)
    pltpu.sync_copy(src_hbm.at[wid], src_v)
    pltpu.sync_copy(dst_hbm.at[wid], dst_v)

    for b in range(NBUF):
        pltpu.async_copy(ys[0].at[src_v.at[b]], bufs.at[b], sems[b])

    for h, (y_hbm, out_hbm) in enumerate(zip(ys, outs)):
        pltpu.sync_copy(zeros_hbm.at[pl.ds(s * RPT, RPT), pl.ds(0, fw)],
                        accum.at[pl.ds(s * RPT, RPT)])
        plsc.subcore_barrier()

        @pl.loop(0, CH, step=NBUF)
        def _grp(i):
            for b in range(NBUF):
                ch = i + b
                pltpu.make_async_copy(y_hbm.at[src_v.at[ch]], bufs.at[b],
                                      sems[b]).wait()
                pltpu.sync_copy(bufs.at[b], accum.at[dst_v.at[ch]], add=True)

                @pl.when(ch + NBUF < CH)
                def _refill():
                    pltpu.async_copy(y_hbm.at[src_v.at[ch + NBUF]],
                                     bufs.at[b], sems[b])

        plsc.subcore_barrier()
        if h + 1 < len(ys):
            for b in range(NBUF):
                pltpu.async_copy(ys[h + 1].at[src_v.at[b]], bufs.at[b],
                                 sems[b])
        pltpu.sync_copy(accum.at[pl.ds(s * RPT, RPT)],
                        out_hbm.at[c, pl.ds(s * RPT, RPT)])


def _make_prop(fw, npair):
    return pl.kernel(
        functools.partial(_prop_body, fw, npair),
        out_type=[jax.ShapeDtypeStruct((NC, NA, fw), jnp.float32)] * npair,
        mesh=plsc.VectorSubcoreMesh(core_axis_name="c", subcore_axis_name="s",
                                    num_cores=NC, num_subcores=NS),
        compiler_params=pltpu.CompilerParams(use_tc_tiling_on_sc=False),
        scratch_types=[
            pltpu.VMEM((CH, K), jnp.int32),
            pltpu.VMEM((CH, K), jnp.int32),
            pltpu.VMEM((NBUF, K, fw), jnp.float32),
            pltpu.VMEM_SHARED((NA, fw), jnp.float32),
        ] + [pltpu.SemaphoreType.DMA] * NBUF,
    )


_prop128 = _make_prop(128, 2)
_prop48 = _make_prop(48, 1)


def _prop256_body(y_hbm, src_hbm, dst_hbm, zeros_hbm, out_hbm,
                  src_v, dst_v, bufs, accum, *sems):
    c = lax.axis_index("c")
    s = lax.axis_index("s")
    pltpu.sync_copy(src_hbm.at[s], src_v)
    pltpu.sync_copy(dst_hbm.at[s], dst_v)
    yh = y_hbm.at[c]

    for b in range(NBUF):
        pltpu.async_copy(yh.at[src_v.at[b]], bufs.at[b], sems[b])
    pltpu.sync_copy(zeros_hbm.at[pl.ds(s * RPT, RPT)],
                    accum.at[pl.ds(s * RPT, RPT)])
    plsc.subcore_barrier()

    @pl.loop(0, CH2, step=NBUF)
    def _grp(i):
        for b in range(NBUF):
            ch = i + b
            pltpu.make_async_copy(yh.at[src_v.at[ch]], bufs.at[b],
                                  sems[b]).wait()
            pltpu.sync_copy(bufs.at[b], accum.at[dst_v.at[ch]], add=True)

            @pl.when(ch + NBUF < CH2)
            def _refill():
                pltpu.async_copy(yh.at[src_v.at[ch + NBUF]],
                                 bufs.at[b], sems[b])

    plsc.subcore_barrier()
    pltpu.sync_copy(accum.at[pl.ds(s * RPT, RPT)],
                    out_hbm.at[c, pl.ds(s * RPT, RPT)])


_prop256 = pl.kernel(
    _prop256_body,
    out_type=jax.ShapeDtypeStruct((NC, NA, 128), jnp.float32),
    mesh=plsc.VectorSubcoreMesh(core_axis_name="c", subcore_axis_name="s",
                                num_cores=NC, num_subcores=NS),
    compiler_params=pltpu.CompilerParams(use_tc_tiling_on_sc=False),
    scratch_types=[
        pltpu.VMEM((CH2, K), jnp.int32),
        pltpu.VMEM((CH2, K), jnp.int32),
        pltpu.VMEM((NBUF, K, 128), jnp.float32),
        pltpu.VMEM_SHARED((NA, 128), jnp.float32),
    ] + [pltpu.SemaphoreType.DMA] * NBUF,
)


def _norm_from(degp_ref):
    deg = degp_ref[0, :, 0] + degp_ref[1, :, 0]
    return lax.rsqrt(jnp.maximum(deg, 1.0))


def _mm1_body(x_ref, w_ref, degp_ref, y_ref):
    norm = _norm_from(degp_ref)
    y = jnp.dot(x_ref[...], w_ref[...],
                preferred_element_type=jnp.float32) * norm[:, None]
    y_ref[0] = y[:, :128]
    y_ref[1] = y[:, 128:]


_mm1 = pl.pallas_call(
    _mm1_body,
    grid=(N_NODES // BM,),
    in_specs=[
        pl.BlockSpec((BM, 256), lambda i: (i, 0)),
        pl.BlockSpec((256, 256), lambda i: (0, 0)),
        pl.BlockSpec((2, BM, 16), lambda i: (0, i, 0)),
    ],
    out_specs=pl.BlockSpec((2, BM, 128), lambda i: (0, i, 0)),
    out_shape=jax.ShapeDtypeStruct((2, N_NODES, 128), jnp.float32),
)


def _mm2_body(p_ref, degp_ref, b_ref, w_ref, y_ref):
    norm = _norm_from(degp_ref)
    nc = norm[:, None]
    ha = jnp.maximum(p_ref[0] * nc + b_ref[0, :128], 0.0)
    hb = jnp.maximum(p_ref[1] * nc + b_ref[0, 128:], 0.0)
    y = (jnp.dot(ha, w_ref[:128, :], preferred_element_type=jnp.float32)
         + jnp.dot(hb, w_ref[128:, :], preferred_element_type=jnp.float32)) * nc
    y_ref[0] = y[:, :128]
    y_ref[1] = y[:, 128:]


_mm2 = pl.pallas_call(
    _mm2_body,
    grid=(N_NODES // BM,),
    in_specs=[
        pl.BlockSpec((2, BM, 128), lambda i: (0, i, 0)),
        pl.BlockSpec((2, BM, 16), lambda i: (0, i, 0)),
        pl.BlockSpec((1, 256), lambda i: (0, 0)),
        pl.BlockSpec((256, 256), lambda i: (0, 0)),
    ],
    out_specs=pl.BlockSpec((2, BM, 128), lambda i: (0, i, 0)),
    out_shape=jax.ShapeDtypeStruct((2, N_NODES, 128), jnp.float32),
)


def _mm3_body(p_ref, degp_ref, b_ref, w_ref, y_ref):
    norm = _norm_from(degp_ref)
    nc = norm[:, None]
    ha = jnp.maximum(p_ref[0] * nc + b_ref[0, :128], 0.0)
    hb = jnp.maximum(p_ref[1] * nc + b_ref[0, 128:], 0.0)
    y_ref[...] = (jnp.dot(ha, w_ref[:128, :], preferred_element_type=jnp.float32)
                  + jnp.dot(hb, w_ref[128:, :],
                            preferred_element_type=jnp.float32)) * nc


_mm3 = pl.pallas_call(
    _mm3_body,
    grid=(N_NODES // BM,),
    in_specs=[
        pl.BlockSpec((2, BM, 128), lambda i: (0, i, 0)),
        pl.BlockSpec((2, BM, 16), lambda i: (0, i, 0)),
        pl.BlockSpec((1, 256), lambda i: (0, 0)),
        pl.BlockSpec((256, 48), lambda i: (0, 0)),
    ],
    out_specs=pl.BlockSpec((BM, 48), lambda i: (i, 0)),
    out_shape=jax.ShapeDtypeStruct((N_NODES, 48), jnp.float32),
)


def _final_body(p_ref, degp_ref, b_ref, o_ref):
    norm = _norm_from(degp_ref)
    o_ref[...] = ((p_ref[0, :, :40] + p_ref[1, :, :40]) * norm[:, None]
                  + b_ref[0])


_final = pl.pallas_call(
    _final_body,
    grid=(N_NODES // BM,),
    in_specs=[
        pl.BlockSpec((2, BM, 48), lambda i: (0, i, 0)),
        pl.BlockSpec((2, BM, 16), lambda i: (0, i, 0)),
        pl.BlockSpec((1, 40), lambda i: (0, 0)),
    ],
    out_specs=pl.BlockSpec((BM, 40), lambda i: (i, 0)),
    out_shape=jax.ShapeDtypeStruct((N_NODES, 40), jnp.float32),
)


def kernel(features, edge_index, W1, b1, W2, b2, W3, b3):
    src_t = edge_index[0].reshape(NW, CH, K)
    dst_t = edge_index[1].reshape(NW, CH, K)
    srcp = edge_index[0].reshape(NS, CH2, K)
    dstp = edge_index[1].reshape(NS, CH2, K)

    ones16 = jnp.ones((K, 16), jnp.float32)
    zeros128 = jnp.zeros((NA, 128), jnp.float32)
    W3p = jnp.pad(W3, ((0, 0), (0, 48 - W3.shape[1])))
    b1r = b1.reshape(1, 256)
    b2r = b2.reshape(1, 256)
    b3r = b3.reshape(1, 40)

    degp = _deg_kernel(dst_t, ones16, zeros128)
    y1 = _mm1(features, W1, degp)
    p1 = _prop256(y1, srcp, dstp, zeros128)
    y2 = _mm2(p1, degp, b1r, W2)
    p2 = _prop256(y2, srcp, dstp, zeros128)
    y3 = _mm3(p2, degp, b2r, W3p)
    (p3,) = _prop48(y3, src_t, dst_t, zeros128)
    return _final(p3, degp, b3r)


# speedup vs baseline: 3.9469x; 1.0267x over previous
import functools

import jax
import jax.numpy as jnp
from jax import lax
from jax.experimental import pallas as pl
from jax.experimental.pallas import tpu as pltpu
from jax.experimental.pallas import tpu_sc as plsc

N_NODES = 10000
N_EDGES = 160000
NC, NS = 2, 16
NW = NC * NS
K = 40
CH = 125
EPT = CH * K
EPW = N_EDGES // NW
NBUF = 5
NA = 10112
RPT = NA // NS

CH2 = N_EDGES // (NS * K)

BM = 2000


def _worker(c, s):
    return s * NC + c


def _deg_body(dst_hbm, ones_hbm, zeros_hbm, out_hbm, idx_v, ones_v, accum, sem):
    c = lax.axis_index("c")
    s = lax.axis_index("s")
    wid = _worker(c, s)
    pltpu.sync_copy(dst_hbm.at[wid], idx_v)
    pltpu.sync_copy(ones_hbm, ones_v)
    pltpu.sync_copy(zeros_hbm.at[pl.ds(s * RPT, RPT), pl.ds(0, 16)],
                    accum.at[pl.ds(s * RPT, RPT)])
    plsc.subcore_barrier()

    @pl.loop(0, CH, step=5)
    def _grp(i):
        for b in range(5):
            pltpu.async_copy(ones_v, accum.at[idx_v.at[i + b]], sem, add=True)
        for b in range(5):
            pltpu.make_async_copy(ones_v, accum.at[idx_v.at[i + b]], sem).wait()

    plsc.subcore_barrier()
    pltpu.sync_copy(accum.at[pl.ds(s * RPT, RPT)],
                    out_hbm.at[c, pl.ds(s * RPT, RPT)])


_deg_kernel = pl.kernel(
    _deg_body,
    out_type=jax.ShapeDtypeStruct((NC, NA, 16), jnp.float32),
    mesh=plsc.VectorSubcoreMesh(core_axis_name="c", subcore_axis_name="s",
                                num_cores=NC, num_subcores=NS),
    compiler_params=pltpu.CompilerParams(use_tc_tiling_on_sc=False),
    scratch_types=[
        pltpu.VMEM((CH, K), jnp.int32),
        pltpu.VMEM((K, 16), jnp.float32),
        pltpu.VMEM_SHARED((NA, 16), jnp.float32),
        pltpu.SemaphoreType.DMA,
    ],
)


def _prop_body(fw, npair, *refs):
    ys = refs[:npair]
    src_hbm, dst_hbm, zeros_hbm = refs[npair:npair + 3]
    outs = refs[npair + 3:2 * npair + 3]
    scr = refs[2 * npair + 3:]
    src_v, dst_v, bufs, accum = scr[:4]
    sems = scr[4:]

    c = lax.axis_index("c")
    s = lax.axis_index("s")
    wid = _worker(c, s)
    pltpu.sync_copy(src_hbm.at[wid], src_v)
    pltpu.sync_copy(dst_hbm.at[wid], dst_v)

    for b in range(NBUF):
        pltpu.async_copy(ys[0].at[src_v.at[b]], bufs.at[b], sems[b])

    for h, (y_hbm, out_hbm) in enumerate(zip(ys, outs)):
        pltpu.sync_copy(zeros_hbm.at[pl.ds(s * RPT, RPT), pl.ds(0, fw)],
                        accum.at[pl.ds(s * RPT, RPT)])
        plsc.subcore_barrier()

        @pl.loop(0, CH, step=NBUF)
        def _grp(i):
            for b in range(NBUF):
                ch = i + b
                pltpu.make_async_copy(y_hbm.at[src_v.at[ch]], bufs.at[b],
                                      sems[b]).wait()
                pltpu.sync_copy(bufs.at[b], accum.at[dst_v.at[ch]], add=True)

                @pl.when(ch + NBUF < CH)
                def _refill():
                    pltpu.async_copy(y_hbm.at[src_v.at[ch + NBUF]],
                                     bufs.at[b], sems[b])

        plsc.subcore_barrier()
        if h + 1 < len(ys):
            for b in range(NBUF):
                pltpu.async_copy(ys[h + 1].at[src_v.at[b]], bufs.at[b],
                                 sems[b])
        pltpu.sync_copy(accum.at[pl.ds(s * RPT, RPT)],
                        out_hbm.at[c, pl.ds(s * RPT, RPT)])


def _make_prop(fw, npair):
    return pl.kernel(
        functools.partial(_prop_body, fw, npair),
        out_type=[jax.ShapeDtypeStruct((NC, NA, fw), jnp.float32)] * npair,
        mesh=plsc.VectorSubcoreMesh(core_axis_name="c", subcore_axis_name="s",
                                    num_cores=NC, num_subcores=NS),
        compiler_params=pltpu.CompilerParams(use_tc_tiling_on_sc=False),
        scratch_types=[
            pltpu.VMEM((CH, K), jnp.int32),
            pltpu.VMEM((CH, K), jnp.int32),
            pltpu.VMEM((NBUF, K, fw), jnp.float32),
            pltpu.VMEM_SHARED((NA, fw), jnp.float32),
        ] + [pltpu.SemaphoreType.DMA] * NBUF,
    )


_prop128 = _make_prop(128, 2)
_prop48 = _make_prop(48, 1)


def _prop256_body(y_hbm, src_hbm, dst_hbm, zeros_hbm, out_hbm,
                  src_v, dst_v, bufs, accum, *sems):
    c = lax.axis_index("c")
    s = lax.axis_index("s")
    pltpu.sync_copy(src_hbm.at[s], src_v)
    pltpu.sync_copy(dst_hbm.at[s], dst_v)
    yh = y_hbm.at[c]

    for b in range(NBUF):
        pltpu.async_copy(yh.at[src_v.at[b]], bufs.at[b], sems[b])
    pltpu.sync_copy(zeros_hbm.at[pl.ds(s * RPT, RPT)],
                    accum.at[pl.ds(s * RPT, RPT)])
    plsc.subcore_barrier()

    @pl.loop(0, CH2, step=NBUF)
    def _grp(i):
        for b in range(NBUF):
            ch = i + b
            pltpu.make_async_copy(yh.at[src_v.at[ch]], bufs.at[b],
                                  sems[b]).wait()
            pltpu.sync_copy(bufs.at[b], accum.at[dst_v.at[ch]], add=True)

            @pl.when(ch + NBUF < CH2)
            def _refill():
                pltpu.async_copy(yh.at[src_v.at[ch + NBUF]],
                                 bufs.at[b], sems[b])

    plsc.subcore_barrier()
    pltpu.sync_copy(accum.at[pl.ds(s * RPT, RPT)],
                    out_hbm.at[c, pl.ds(s * RPT, RPT)])


_prop256 = pl.kernel(
    _prop256_body,
    out_type=jax.ShapeDtypeStruct((NC, NA, 128), jnp.float32),
    mesh=plsc.VectorSubcoreMesh(core_axis_name="c", subcore_axis_name="s",
                                num_cores=NC, num_subcores=NS),
    compiler_params=pltpu.CompilerParams(use_tc_tiling_on_sc=False),
    scratch_types=[
        pltpu.VMEM((CH2, K), jnp.int32),
        pltpu.VMEM((CH2, K), jnp.int32),
        pltpu.VMEM((NBUF, K, 128), jnp.float32),
        pltpu.VMEM_SHARED((NA, 128), jnp.float32),
    ] + [pltpu.SemaphoreType.DMA] * NBUF,
)


def _norm_from(degp_ref):
    deg = degp_ref[0, :, 0] + degp_ref[1, :, 0]
    return lax.rsqrt(jnp.maximum(deg, 1.0))


def _mm1_body(x_ref, w_ref, degp_ref, y_ref):
    norm = _norm_from(degp_ref)
    y = jnp.dot(x_ref[...], w_ref[...],
                preferred_element_type=jnp.float32) * norm[:, None]
    y_ref[0] = y[:, :128]
    y_ref[1] = y[:, 128:]


_mm1 = pl.pallas_call(
    _mm1_body,
    grid=(N_NODES // BM,),
    in_specs=[
        pl.BlockSpec((BM, 256), lambda i: (i, 0)),
        pl.BlockSpec((256, 256), lambda i: (0, 0)),
        pl.BlockSpec((2, BM, 16), lambda i: (0, i, 0)),
    ],
    out_specs=pl.BlockSpec((2, BM, 128), lambda i: (0, i, 0)),
    out_shape=jax.ShapeDtypeStruct((2, N_NODES, 128), jnp.float32),
)


def _mm2_body(p_ref, degp_ref, b_ref, w_ref, y_ref):
    norm = _norm_from(degp_ref)
    nc = norm[:, None]
    ha = jnp.maximum(p_ref[0] * nc + b_ref[0, :128], 0.0)
    hb = jnp.maximum(p_ref[1] * nc + b_ref[0, 128:], 0.0)
    y = (jnp.dot(ha, w_ref[:128, :], preferred_element_type=jnp.float32)
         + jnp.dot(hb, w_ref[128:, :], preferred_element_type=jnp.float32)) * nc
    y_ref[0] = y[:, :128]
    y_ref[1] = y[:, 128:]


_mm2 = pl.pallas_call(
    _mm2_body,
    grid=(N_NODES // BM,),
    in_specs=[
        pl.BlockSpec((2, BM, 128), lambda i: (0, i, 0)),
        pl.BlockSpec((2, BM, 16), lambda i: (0, i, 0)),
        pl.BlockSpec((1, 256), lambda i: (0, 0)),
        pl.BlockSpec((256, 256), lambda i: (0, 0)),
    ],
    out_specs=pl.BlockSpec((2, BM, 128), lambda i: (0, i, 0)),
    out_shape=jax.ShapeDtypeStruct((2, N_NODES, 128), jnp.float32),
)


def _mm3_body(p_ref, degp_ref, b_ref, w_ref, y_ref):
    norm = _norm_from(degp_ref)
    nc = norm[:, None]
    ha = jnp.maximum(p_ref[0] * nc + b_ref[0, :128], 0.0)
    hb = jnp.maximum(p_ref[1] * nc + b_ref[0, 128:], 0.0)
    y_ref[...] = (jnp.dot(ha, w_ref[:128, :], preferred_element_type=jnp.float32)
                  + jnp.dot(hb, w_ref[128:, :],
                            preferred_element_type=jnp.float32)) * nc


_mm3 = pl.pallas_call(
    _mm3_body,
    grid=(N_NODES // BM,),
    in_specs=[
        pl.BlockSpec((2, BM, 128), lambda i: (0, i, 0)),
        pl.BlockSpec((2, BM, 16), lambda i: (0, i, 0)),
        pl.BlockSpec((1, 256), lambda i: (0, 0)),
        pl.BlockSpec((256, 48), lambda i: (0, 0)),
    ],
    out_specs=pl.BlockSpec((BM, 48), lambda i: (i, 0)),
    out_shape=jax.ShapeDtypeStruct((N_NODES, 48), jnp.float32),
)


def _final_body(p_ref, degp_ref, b_ref, o_ref):
    norm = _norm_from(degp_ref)
    o_ref[...] = ((p_ref[0, :, :40] + p_ref[1, :, :40]) * norm[:, None]
                  + b_ref[0])


_final = pl.pallas_call(
    _final_body,
    grid=(N_NODES // BM,),
    in_specs=[
        pl.BlockSpec((2, BM, 48), lambda i: (0, i, 0)),
        pl.BlockSpec((2, BM, 16), lambda i: (0, i, 0)),
        pl.BlockSpec((1, 40), lambda i: (0, 0)),
    ],
    out_specs=pl.BlockSpec((BM, 40), lambda i: (i, 0)),
    out_shape=jax.ShapeDtypeStruct((N_NODES, 40), jnp.float32),
)


def kernel(features, edge_index, W1, b1, W2, b2, W3, b3):
    src_t = edge_index[0].reshape(NW, CH, K)
    dst_t = edge_index[1].reshape(NW, CH, K)
    srcp = edge_index[0].reshape(NS, CH2, K)
    dstp = edge_index[1].reshape(NS, CH2, K)

    ones16 = jnp.ones((K, 16), jnp.float32)
    zeros128 = jnp.zeros((NA, 128), jnp.float32)
    W3p = jnp.pad(W3, ((0, 0), (0, 48 - W3.shape[1])))
    b1r = b1.reshape(1, 256)
    b2r = b2.reshape(1, 256)
    b3r = b3.reshape(1, 40)

    degp = _deg_kernel(dst_t, ones16, zeros128)
    y1 = _mm1(features, W1, degp)
    p1 = _prop256(y1, srcp, dstp, zeros128)
    y2 = _mm2(p1, degp, b1r, W2)
    p2 = _prop256(y2, srcp, dstp, zeros128)
    y3 = _mm3(p2, degp, b2r, W3p)
    (p3,) = _prop48(y3, src_t, dst_t, zeros128)
    return _final(p3, degp, b3r)
